# baseline (speedup 1.0000x reference)
import functools

import jax
import jax.numpy as jnp
from jax import lax
from jax.experimental import pallas as pl
from jax.experimental.pallas import tpu as pltpu
from jax.experimental.pallas import tpu_sc as plsc

LANES = 16


def kernel(input_ids, token_table, pos_table):
    B, S = input_ids.shape
    V, D = token_table.shape
    NW = 32
    CH = 16
    NB = B // (NW * CH)

    ids_w = input_ids.astype(jnp.int32).reshape(NW, NB * CH, S).transpose(0, 2, 1)
    pos_r = jnp.pad(pos_table, ((0, 1), (0, 0))).reshape(S + 1, 8, 128)

    mesh = plsc.VectorSubcoreMesh(core_axis_name="c", subcore_axis_name="s")

    @functools.partial(
        pl.kernel,
        mesh=mesh,
        out_type=jax.ShapeDtypeStruct((S, B, D), jnp.float32),
        scratch_types=[
            pltpu.VMEM((2, 8, 128), jnp.float32),
            pltpu.VMEM((S, NB * CH), jnp.int32),
            pltpu.VMEM((CH, D), jnp.float32),
            pltpu.VMEM((CH, D), jnp.float32),
            pltpu.VMEM((CH, D), jnp.float32),
            pltpu.VMEM((CH, D), jnp.float32),
            pltpu.SemaphoreType.DMA,
            pltpu.SemaphoreType.DMA,
            pltpu.SemaphoreType.DMA,
            pltpu.SemaphoreType.DMA,
            pltpu.SemaphoreType.DMA,
            pltpu.SemaphoreType.DMA,
            pltpu.SemaphoreType.DMA,
            pltpu.SemaphoreType.DMA,
            pltpu.SemaphoreType.DMA,
        ],
    )
    def k(idx_hbm, tok_hbm, pos_hbm, out_hbm, pos_v, idx_v,
          b0, b1, b2, b3, g0, g1, g2, g3, s0_, s1_, s2_, s3_, psem):
        bufs = [b0, b1, b2, b3]
        gsems = [g0, g1, g2, g3]
        ssems = [s0_, s1_, s2_, s3_]
        c = lax.axis_index("c")
        s = lax.axis_index("s")
        wid = s * 2 + c
        col0 = wid * NB * CH
        pltpu.sync_copy(idx_hbm.at[wid], idx_v)
        pltpu.sync_copy(pos_hbm.at[0], pos_v.at[0])

        def gather(srow, j, slot):
            pltpu.async_copy(
                tok_hbm.at[idx_v.at[srow, pl.ds(CH * j, CH)]],
                bufs[slot], gsems[slot])

        def gather_wait(srow, j, slot):
            pltpu.make_async_copy(
                tok_hbm.at[idx_v.at[srow, pl.ds(CH * j, CH)]],
                bufs[slot], gsems[slot]).wait()

        def scatter(srow, j, slot):
            pltpu.async_copy(
                bufs[slot], out_hbm.at[srow, pl.ds(col0 + CH * j, CH), :],
                ssems[slot])

        def scatter_wait(srow, j, slot):
            pltpu.make_async_copy(
                bufs[slot], out_hbm.at[srow, pl.ds(col0 + CH * j, CH), :],
                ssems[slot]).wait()

        def pos_fetch(srow, pp):
            pltpu.async_copy(pos_hbm.at[srow], pos_v.at[pp], psem)

        def pos_wait(srow, pp):
            pltpu.make_async_copy(
                pos_hbm.at[srow], pos_v.at[pp], psem).wait()

        def add_chunk(slot, pp):
            @plsc.parallel_loop(0, D // LANES, unroll=4)
            def add_j(jj):
                t = jj // 8
                sl = pl.ds((jj % 8) * LANES, LANES)
                v = pos_v[pp, t, sl]
                slb = pl.ds(jj * LANES, LANES)
                for r in range(CH):
                    plsc.addupdate(bufs[slot].at[r, slb], v)

        gather(0, 0, 0)
        gather(0, 1, 1)

        def dround(dd, carry):
            for q in range(4):
                rd = 2 * dd + q // 2
                j = q % 2
                slot = q
                pslot = (q - 2) % 4
                pp = q // 2
                if q == 0:
                    @pl.when(dd > 0)
                    def _():
                        pos_wait(rd, pp)
                    pos_fetch(rd + 1, 1 - pp)
                if q == 2:
                    pos_wait(rd, pp)
                    pos_fetch(rd + 1, 1 - pp)
                rdm2 = 2 * dd + (q - 2) // 2
                if q < 2:
                    @pl.when(dd > 0)
                    def _():
                        scatter_wait(rdm2, j, pslot)
                else:
                    scatter_wait(rdm2, j, pslot)
                rdp2 = 2 * dd + (q + 2) // 2
                gather(rdp2, j, pslot)
                gather_wait(rd, j, slot)
                add_chunk(slot, pp)
                scatter(rd, j, slot)
            return carry

        lax.fori_loop(0, S // 2, dround, 0)
        pos_wait(S - 1, 0)
        for j in range(2):
            scatter_wait(S - 2, j, j + 2)
            gather_wait(S - 1, j, j)
            add_chunk(j, 0)
            scatter(S - 1, j, j)
        scatter_wait(S - 1, 0, 0)
        scatter_wait(S - 1, 1, 1)

    out = k(ids_w, token_table, pos_r)
    return jnp.transpose(out, (1, 0, 2))
